# baseline (speedup 1.0000x reference)
import functools

import jax
import jax.numpy as jnp
from jax import lax
from jax.experimental import pallas as pl
from jax.experimental.pallas import tpu as pltpu
from jax.experimental.pallas import tpu_sc as plsc

D = 128
DW = D // 2
B = 1024
S1 = 32
S2 = 16
M1 = B * S1
NCLS = 64
NC, NS = 2, 16
NW = NC * NS
L = 16

CH = 128
G0_PW = B // NW
G1_PW = M1 // NW
G1_CHUNKS = G1_PW // CH
S2_OUT_PC = CH // S2
S2_CHUNKS = G1_PW // S2_OUT_PC

_HIMASK = -65536


def _sc_gather(embq, t0r, t1r, t2r):
    mesh = plsc.VectorSubcoreMesh(core_axis_name="c", subcore_axis_name="s")

    @functools.partial(
        pl.kernel,
        mesh=mesh,
        compiler_params=pltpu.CompilerParams(use_tc_tiling_on_sc=False),
        out_type=[
            jax.ShapeDtypeStruct((B, DW), jnp.int32),
            jax.ShapeDtypeStruct((M1, DW), jnp.int32),
            jax.ShapeDtypeStruct((M1, D), jnp.float32),
        ],
        scratch_types=[
            pltpu.VMEM((G0_PW,), jnp.int32),
            pltpu.VMEM((G1_CHUNKS, CH), jnp.int32),
            pltpu.VMEM((S2_CHUNKS, CH), jnp.int32),
            pltpu.VMEM((CH, DW), jnp.int32),
            pltpu.VMEM((CH, DW), jnp.int32),
            pltpu.VMEM((S2_OUT_PC, D), jnp.float32),
            pltpu.VMEM((S2_OUT_PC, D), jnp.float32),
            pltpu.SemaphoreType.DMA,
            pltpu.SemaphoreType.DMA,
            pltpu.SemaphoreType.DMA,
            pltpu.SemaphoreType.DMA,
        ],
    )
    def k(emb_h, t0_h, t1_h, t2_h, g0_h, g1_h, s2_h,
          idx0_v, idx1_v, idx2_v, rows_a, rows_b, acc_a, acc_b,
          sem_ga, sem_gb, sem_wa, sem_wb):
        wid = lax.axis_index("s") * NC + lax.axis_index("c")
        b1 = wid * G1_PW

        pltpu.sync_copy(t0_h.at[wid], idx0_v)
        pltpu.sync_copy(t1_h.at[wid], idx1_v)
        pltpu.sync_copy(t2_h.at[wid], idx2_v)

        b0 = wid * G0_PW
        pltpu.async_copy(emb_h.at[idx0_v], rows_a.at[pl.ds(0, G0_PW)], sem_ga).wait()
        pltpu.sync_copy(rows_a.at[pl.ds(0, G0_PW)], g0_h.at[pl.ds(b0, G0_PW)])

        def g1_gather(c, rows, sem):
            return pltpu.make_async_copy(emb_h.at[idx1_v.at[c]], rows, sem)

        def g1_write(c, rows, sem):
            return pltpu.make_async_copy(rows, g1_h.at[pl.ds(b1 + c * CH, CH)], sem)

        g1_gather(0, rows_a, sem_ga).start()
        g1_gather(1, rows_b, sem_gb).start()

        def g1_loop(cc, carry):
            c0 = 2 * cc
            c1 = c0 + 1
            g1_gather(c0, rows_a, sem_ga).wait()
            g1_write(c0, rows_a, sem_wa).start()
            g1_gather(c1, rows_b, sem_gb).wait()
            g1_write(c1, rows_b, sem_wb).start()

            @pl.when(cc < G1_CHUNKS // 2 - 1)
            def _():
                g1_write(c0, rows_a, sem_wa).wait()
                g1_gather(c0 + 2, rows_a, sem_ga).start()
                g1_write(c1, rows_b, sem_wb).wait()
                g1_gather(c1 + 2, rows_b, sem_gb).start()

            return carry

        lax.fori_loop(0, G1_CHUNKS // 2, g1_loop, 0)
        g1_write(0, rows_a, sem_wa).wait()
        g1_write(0, rows_b, sem_wb).wait()

        def s2_gather(c, rows, sem):
            return pltpu.make_async_copy(emb_h.at[idx2_v.at[c]], rows, sem)

        def s2_write(c, acc, sem):
            return pltpu.make_async_copy(
                acc, s2_h.at[pl.ds(b1 + c * S2_OUT_PC, S2_OUT_PC)], sem)

        def _tree(vs):
            while len(vs) > 1:
                vs = [a + b for a, b in zip(vs[::2], vs[1::2])]
            return vs[0]

        def s2_reduce(rows, acc):
            def red_j(j, carry2):
                base = j * S2
                for lg in range(DW // L):
                    sl = pl.ds(lg * L, L)
                    los, his = [], []
                    for s in range(S2):
                        xi = rows[base + s, sl]
                        los.append(lax.bitcast_convert_type(xi << 16, jnp.float32))
                        his.append(lax.bitcast_convert_type(xi & _HIMASK, jnp.float32))
                    acc[j, sl] = _tree(los)
                    acc[j, pl.ds(DW + lg * L, L)] = _tree(his)
                return carry2

            lax.fori_loop(0, S2_OUT_PC, red_j, 0)

        s2_gather(0, rows_a, sem_ga).start()

        def s2_loop(cc, carry):
            c0 = 2 * cc
            c1 = c0 + 1
            s2_gather(c1, rows_b, sem_gb).start()
            s2_gather(c0, rows_a, sem_ga).wait()

            @pl.when(cc > 0)
            def _():
                s2_write(c0, acc_a, sem_wa).wait()

            s2_reduce(rows_a, acc_a)
            s2_write(c0, acc_a, sem_wa).start()

            @pl.when(cc < S2_CHUNKS // 2 - 1)
            def _():
                s2_gather(c0 + 2, rows_a, sem_ga).start()

            s2_gather(c1, rows_b, sem_gb).wait()

            @pl.when(cc > 0)
            def _():
                s2_write(c1, acc_b, sem_wb).wait()

            s2_reduce(rows_b, acc_b)
            s2_write(c1, acc_b, sem_wb).start()
            return carry

        lax.fori_loop(0, S2_CHUNKS // 2, s2_loop, 0)
        s2_write(0, acc_a, sem_wa).wait()
        s2_write(0, acc_b, sem_wb).wait()

    return k(embq, t0r, t1r, t2r)


TCB = 2048
NSTEP = M1 // TCB
GPB = TCB // S1


def _unpack(xi):
    lo = lax.bitcast_convert_type(xi << 16, jnp.float32)
    hi = lax.bitcast_convert_type(xi & _HIMASK, jnp.float32)
    return lo, hi


def _tc_body(g0_r, g1_r, s2_r, w1ae_r, w1ao_r, w1be_r, w1bo_r, w1bs_r,
             w2a_r, w2b_r, wout_r, out_r, x1lo_r, x1hi_r, b1_r):
    i = pl.program_id(0)
    g1lo, g1hi = _unpack(g1_r[...])
    h = jnp.dot(g1lo, w1ae_r[...], preferred_element_type=jnp.float32)
    h = h + jnp.dot(g1hi, w1ao_r[...], preferred_element_type=jnp.float32)
    h = h + jnp.dot(s2_r[...], w1bs_r[...], preferred_element_type=jnp.float32)
    h = jnp.maximum(h, 0.0)
    b1_r[pl.ds(i * GPB, GPB), :] = h.reshape(GPB, S1, D).sum(axis=1) * (1.0 / S1)
    x1lo_r[pl.ds(i * GPB, GPB), :] = g1lo.reshape(GPB, S1, DW).sum(axis=1) * (1.0 / S1)
    x1hi_r[pl.ds(i * GPB, GPB), :] = g1hi.reshape(GPB, S1, DW).sum(axis=1) * (1.0 / S1)

    @pl.when(i == NSTEP - 1)
    def _():
        g0lo, g0hi = _unpack(g0_r[...])
        a = jnp.dot(g0lo, w1ae_r[...], preferred_element_type=jnp.float32)
        a = a + jnp.dot(g0hi, w1ao_r[...], preferred_element_type=jnp.float32)
        a = a + jnp.dot(x1lo_r[...], w1be_r[...], preferred_element_type=jnp.float32)
        a = a + jnp.dot(x1hi_r[...], w1bo_r[...], preferred_element_type=jnp.float32)
        a = jnp.maximum(a, 0.0)
        c = jnp.dot(a, w2a_r[...], preferred_element_type=jnp.float32)
        c = c + jnp.dot(b1_r[...], w2b_r[...], preferred_element_type=jnp.float32)
        c = jnp.maximum(c, 0.0)
        out_r[...] = jnp.dot(c, wout_r[...], preferred_element_type=jnp.float32)


def _tc_dense(g0, g1, s2, w1ae, w1ao, w1be, w1bo, w1bs, w2a, w2b, woutT):
    full = lambda shape: pl.BlockSpec(shape, lambda i: (0, 0))
    return pl.pallas_call(
        _tc_body,
        grid=(NSTEP,),
        in_specs=[
            full((B, DW)),
            pl.BlockSpec((TCB, DW), lambda i: (i, 0)),
            pl.BlockSpec((TCB, D), lambda i: (i, 0)),
            full((DW, D)), full((DW, D)), full((DW, D)), full((DW, D)),
            full((D, D)),
            full((D, D)), full((D, D)),
            full((D, NCLS)),
        ],
        out_specs=full((B, NCLS)),
        out_shape=jax.ShapeDtypeStruct((B, NCLS), jnp.float32),
        scratch_shapes=[
            pltpu.VMEM((B, DW), jnp.float32),
            pltpu.VMEM((B, DW), jnp.float32),
            pltpu.VMEM((B, D), jnp.float32),
        ],
    )(g0, g1, s2, w1ae, w1ao, w1be, w1bo, w1bs, w2a, w2b, woutT)


def kernel(T0, T1, T2, emb, W1, W2, Wout):
    t0r = T0.astype(jnp.int32).reshape(NW, G0_PW)
    t1r = T1.reshape(-1).astype(jnp.int32).reshape(NW, G1_CHUNKS, CH)
    t2r = T2.reshape(-1).astype(jnp.int32).reshape(NW, S2_CHUNKS, CH)
    embq = lax.bitcast_convert_type(
        emb.astype(jnp.bfloat16).reshape(emb.shape[0], DW, 2), jnp.int32)
    w1a = W1[:, :D].T
    w1b = W1[:, D:].T
    w1ae, w1ao = w1a[0::2, :], w1a[1::2, :]
    w1be, w1bo = w1b[0::2, :], w1b[1::2, :]
    w1bs = jnp.concatenate([w1be, w1bo], axis=0) * (1.0 / S2)
    w2a = W2[:, :D].T
    w2b = W2[:, D:].T
    woutT = Wout.T
    g0, g1, s2 = _sc_gather(embq, t0r, t1r, t2r)
    return _tc_dense(g0, g1, s2, w1ae, w1ao, w1be, w1bo, w1bs, w2a, w2b, woutT)


# speedup vs baseline: 2.1428x; 2.1428x over previous
import functools

import jax
import jax.numpy as jnp
from jax import lax
from jax.experimental import pallas as pl
from jax.experimental.pallas import tpu as pltpu
from jax.experimental.pallas import tpu_sc as plsc

D = 128
DW = D // 2
B = 1024
S1 = 32
S2 = 16
M1 = B * S1
NCLS = 64
NC, NS = 2, 16
NW = NC * NS
L = 16

CH = 128
G0_PW = B // NW
G1_PW = M1 // NW
G1_CHUNKS = G1_PW // CH
S2_OUT_PC = CH // S2
S2_CHUNKS = G1_PW // S2_OUT_PC

_HIMASK = -65536


def _sc_gather(embq, t0r, t1r, t2r):
    mesh = plsc.VectorSubcoreMesh(core_axis_name="c", subcore_axis_name="s")

    @functools.partial(
        pl.kernel,
        mesh=mesh,
        compiler_params=pltpu.CompilerParams(use_tc_tiling_on_sc=False),
        out_type=[
            jax.ShapeDtypeStruct((B, D), jnp.float32),
            jax.ShapeDtypeStruct((M1, D), jnp.float32),
            jax.ShapeDtypeStruct((M1, D), jnp.float32),
        ],
        scratch_types=[
            pltpu.VMEM((G0_PW,), jnp.int32),
            pltpu.VMEM((G1_CHUNKS, CH), jnp.int32),
            pltpu.VMEM((S2_CHUNKS, CH), jnp.int32),
            pltpu.VMEM((CH, DW), jnp.int32),
            pltpu.VMEM((CH, DW), jnp.int32),
            pltpu.VMEM((CH, D), jnp.float32),
            pltpu.VMEM((CH, D), jnp.float32),
            pltpu.VMEM((S2_OUT_PC, D), jnp.float32),
            pltpu.VMEM((S2_OUT_PC, D), jnp.float32),
            pltpu.SemaphoreType.DMA,
            pltpu.SemaphoreType.DMA,
            pltpu.SemaphoreType.DMA,
            pltpu.SemaphoreType.DMA,
        ],
    )
    def k(emb_h, t0_h, t1_h, t2_h, g0_h, g1_h, s2_h,
          idx0_v, idx1_v, idx2_v, rows_a, rows_b, stg_a, stg_b, acc_a, acc_b,
          sem_ga, sem_gb, sem_wa, sem_wb):
        wid = lax.axis_index("s") * NC + lax.axis_index("c")
        b1 = wid * G1_PW

        pltpu.sync_copy(t0_h.at[wid // 4, pl.ds((wid % 4) * G0_PW, G0_PW)], idx0_v)
        pltpu.sync_copy(t1_h.at[wid], idx1_v)
        pltpu.sync_copy(t2_h.at[wid], idx2_v)

        def unpack_rows(rows, stg, nrows):
            def up_r(r, carry):
                for lg in range(DW // L):
                    sl = pl.ds(lg * L, L)
                    xi = rows[r, sl]
                    stg[r, sl] = lax.bitcast_convert_type(xi << 16, jnp.float32)
                    stg[r, pl.ds(DW + lg * L, L)] = lax.bitcast_convert_type(
                        xi & _HIMASK, jnp.float32)
                return carry

            lax.fori_loop(0, nrows, up_r, 0)

        b0 = wid * G0_PW
        pltpu.async_copy(emb_h.at[idx0_v], rows_a.at[pl.ds(0, G0_PW)], sem_ga).wait()
        unpack_rows(rows_a, stg_a, G0_PW)
        pltpu.sync_copy(stg_a.at[pl.ds(0, G0_PW)], g0_h.at[pl.ds(b0, G0_PW)])

        def g1_gather(c, rows, sem):
            return pltpu.make_async_copy(emb_h.at[idx1_v.at[c]], rows, sem)

        def g1_write(c, stg, sem):
            return pltpu.make_async_copy(stg, g1_h.at[pl.ds(b1 + c * CH, CH)], sem)

        g1_gather(0, rows_a, sem_ga).start()
        g1_gather(1, rows_b, sem_gb).start()

        def g1_loop(cc, carry):
            c0 = 2 * cc
            c1 = c0 + 1
            g1_gather(c0, rows_a, sem_ga).wait()

            @pl.when(cc > 0)
            def _():
                g1_write(c0, stg_a, sem_wa).wait()

            unpack_rows(rows_a, stg_a, CH)
            g1_write(c0, stg_a, sem_wa).start()

            @pl.when(cc < G1_CHUNKS // 2 - 1)
            def _():
                g1_gather(c0 + 2, rows_a, sem_ga).start()

            g1_gather(c1, rows_b, sem_gb).wait()

            @pl.when(cc > 0)
            def _():
                g1_write(c1, stg_b, sem_wb).wait()

            unpack_rows(rows_b, stg_b, CH)
            g1_write(c1, stg_b, sem_wb).start()

            @pl.when(cc < G1_CHUNKS // 2 - 1)
            def _():
                g1_gather(c1 + 2, rows_b, sem_gb).start()

            return carry

        lax.fori_loop(0, G1_CHUNKS // 2, g1_loop, 0)
        g1_write(0, stg_a, sem_wa).wait()
        g1_write(0, stg_b, sem_wb).wait()

        def s2_gather(c, rows, sem):
            return pltpu.make_async_copy(emb_h.at[idx2_v.at[c]], rows, sem)

        def s2_write(c, acc, sem):
            return pltpu.make_async_copy(
                acc, s2_h.at[pl.ds(b1 + c * S2_OUT_PC, S2_OUT_PC)], sem)

        def _tree(vs):
            while len(vs) > 1:
                vs = [a + b for a, b in zip(vs[::2], vs[1::2])]
            return vs[0]

        def s2_reduce(rows, acc):
            def red_j(j, carry2):
                base = j * S2
                for lg in range(DW // L):
                    sl = pl.ds(lg * L, L)
                    los, his = [], []
                    for s in range(S2):
                        xi = rows[base + s, sl]
                        los.append(lax.bitcast_convert_type(xi << 16, jnp.float32))
                        his.append(lax.bitcast_convert_type(xi & _HIMASK, jnp.float32))
                    acc[j, sl] = _tree(los)
                    acc[j, pl.ds(DW + lg * L, L)] = _tree(his)
                return carry2

            lax.fori_loop(0, S2_OUT_PC, red_j, 0)

        s2_gather(0, rows_a, sem_ga).start()

        def s2_loop(cc, carry):
            c0 = 2 * cc
            c1 = c0 + 1
            s2_gather(c1, rows_b, sem_gb).start()
            s2_gather(c0, rows_a, sem_ga).wait()

            @pl.when(cc > 0)
            def _():
                s2_write(c0, acc_a, sem_wa).wait()

            s2_reduce(rows_a, acc_a)
            s2_write(c0, acc_a, sem_wa).start()

            @pl.when(cc < S2_CHUNKS // 2 - 1)
            def _():
                s2_gather(c0 + 2, rows_a, sem_ga).start()

            s2_gather(c1, rows_b, sem_gb).wait()

            @pl.when(cc > 0)
            def _():
                s2_write(c1, acc_b, sem_wb).wait()

            s2_reduce(rows_b, acc_b)
            s2_write(c1, acc_b, sem_wb).start()
            return carry

        lax.fori_loop(0, S2_CHUNKS // 2, s2_loop, 0)
        s2_write(0, acc_a, sem_wa).wait()
        s2_write(0, acc_b, sem_wb).wait()

    return k(embq, t0r, t1r, t2r)


TCB = 2048
NSTEP = M1 // TCB
GPB = TCB // S1


def _tc_body(g0_r, g1_r, s2_r, w1a_r, w1b_r, w1bs_r, w2a_r, w2b_r, wout_r,
             out_r, x1_r, b1_r):
    i = pl.program_id(0)
    g1b = g1_r[...]
    h = jnp.dot(g1b, w1a_r[...], preferred_element_type=jnp.float32)
    h = h + jnp.dot(s2_r[...], w1bs_r[...], preferred_element_type=jnp.float32)
    h = jnp.maximum(h, 0.0)
    b1_r[pl.ds(i * GPB, GPB), :] = h.reshape(GPB, S1, D).sum(axis=1) * (1.0 / S1)
    x1_r[pl.ds(i * GPB, GPB), :] = g1b.reshape(GPB, S1, D).sum(axis=1) * (1.0 / S1)

    @pl.when(i == NSTEP - 1)
    def _():
        a = jnp.dot(g0_r[...], w1a_r[...], preferred_element_type=jnp.float32)
        a = a + jnp.dot(x1_r[...], w1b_r[...], preferred_element_type=jnp.float32)
        a = jnp.maximum(a, 0.0)
        c = jnp.dot(a, w2a_r[...], preferred_element_type=jnp.float32)
        c = c + jnp.dot(b1_r[...], w2b_r[...], preferred_element_type=jnp.float32)
        c = jnp.maximum(c, 0.0)
        out_r[...] = jnp.dot(c, wout_r[...], preferred_element_type=jnp.float32)


def _tc_dense(g0, g1, s2, w1a, w1b, w1bs, w2a, w2b, woutT):
    full = lambda shape: pl.BlockSpec(shape, lambda i: (0, 0))
    return pl.pallas_call(
        _tc_body,
        grid=(NSTEP,),
        in_specs=[
            full((B, D)),
            pl.BlockSpec((TCB, D), lambda i: (i, 0)),
            pl.BlockSpec((TCB, D), lambda i: (i, 0)),
            full((D, D)), full((D, D)), full((D, D)),
            full((D, D)), full((D, D)),
            full((D, NCLS)),
        ],
        out_specs=full((B, NCLS)),
        out_shape=jax.ShapeDtypeStruct((B, NCLS), jnp.float32),
        scratch_shapes=[
            pltpu.VMEM((B, D), jnp.float32),
            pltpu.VMEM((B, D), jnp.float32),
        ],
    )(g0, g1, s2, w1a, w1b, w1bs, w2a, w2b, woutT)


def kernel(T0, T1, T2, emb, W1, W2, Wout):
    t0r = T0.astype(jnp.int32).reshape(B // CH, CH)
    t1r = T1.reshape(-1).astype(jnp.int32).reshape(NW, G1_CHUNKS, CH)
    t2r = T2.reshape(-1).astype(jnp.int32).reshape(NW, S2_CHUNKS, CH)
    u = lax.bitcast_convert_type(emb, jnp.int32)
    bf16r = lambda x: (x + 0x7FFF + ((x >> 16) & 1)) >> 16
    embq = (bf16r(u[:, :DW]) & 0xFFFF) | (bf16r(u[:, DW:]) << 16)
    w1a = W1[:, :D].T
    w1b = W1[:, D:].T
    w1bs = w1b * (1.0 / S2)
    w2a = W2[:, :D].T
    w2b = W2[:, D:].T
    woutT = Wout.T
    g0, g1, s2 = _sc_gather(embq, t0r, t1r, t2r)
    return _tc_dense(g0, g1, s2, w1a, w1b, w1bs, w2a, w2b, woutT)


# speedup vs baseline: 2.4915x; 1.1627x over previous
import functools

import jax
import jax.numpy as jnp
from jax import lax
from jax.experimental import pallas as pl
from jax.experimental.pallas import tpu as pltpu
from jax.experimental.pallas import tpu_sc as plsc

D = 128
DW = D // 2
B = 1024
S1 = 32
S2 = 16
M1 = B * S1
NCLS = 64
NC, NS = 2, 16
NW = NC * NS
L = 16

CH = 128
G0_PW = B // NW
G1_PW = M1 // NW
G1_CHUNKS = G1_PW // CH
S2_OUT_PC = CH // S2
S2_CHUNKS = G1_PW // S2_OUT_PC

_HIMASK = -65536


PK_PW = 100000 // NW
PK_CH = 125
PK_NCH = PK_PW // PK_CH


def _sc_gather01(emb, t0r, t1r):
    mesh = plsc.VectorSubcoreMesh(core_axis_name="c", subcore_axis_name="s")

    @functools.partial(
        pl.kernel,
        mesh=mesh,
        compiler_params=pltpu.CompilerParams(use_tc_tiling_on_sc=False),
        out_type=[
            jax.ShapeDtypeStruct((B, D), jnp.float32),
            jax.ShapeDtypeStruct((M1, D), jnp.float32),
            jax.ShapeDtypeStruct((100000, DW), jnp.int32),
        ],
        scratch_types=[
            pltpu.VMEM((G0_PW,), jnp.int32),
            pltpu.VMEM((G1_CHUNKS, CH), jnp.int32),
            pltpu.VMEM((CH, D), jnp.float32),
            pltpu.VMEM((CH, D), jnp.float32),
            pltpu.VMEM((PK_CH, DW), jnp.int32),
            pltpu.VMEM((PK_CH, DW), jnp.int32),
            pltpu.SemaphoreType.DMA,
            pltpu.SemaphoreType.DMA,
            pltpu.SemaphoreType.DMA,
            pltpu.SemaphoreType.DMA,
        ],
    )
    def k(emb_h, t0_h, t1_h, g0_h, g1_h, embq_h,
          idx0_v, idx1_v, rows_a, rows_b, pout_a, pout_b,
          sem_ga, sem_gb, sem_wa, sem_wb):
        wid = lax.axis_index("s") * NC + lax.axis_index("c")
        b1 = wid * G1_PW

        pltpu.sync_copy(t0_h.at[wid // 4, pl.ds((wid % 4) * G0_PW, G0_PW)], idx0_v)
        pltpu.sync_copy(t1_h.at[wid], idx1_v)

        bp = wid * PK_PW

        def pk_in(c, buf, sem):
            return pltpu.make_async_copy(
                emb_h.at[pl.ds(bp + c * PK_CH, PK_CH)], buf.at[pl.ds(0, PK_CH)], sem)

        def pk_out(c, pbuf, sem):
            return pltpu.make_async_copy(
                pbuf, embq_h.at[pl.ds(bp + c * PK_CH, PK_CH)], sem)

        def pk_pack(buf, pbuf):
            def pk_r(r, carry):
                for lg in range(DW // L):
                    sl = pl.ds(lg * L, L)
                    x = lax.bitcast_convert_type(buf[r, sl], jnp.int32)
                    y = lax.bitcast_convert_type(buf[r, pl.ds(DW + lg * L, L)], jnp.int32)
                    x2 = x + 32767 + ((x >> 16) & 1)
                    y2 = y + 32767 + ((y >> 16) & 1)
                    pbuf[r, sl] = ((x2 >> 16) & 0xFFFF) | (y2 & _HIMASK)
                return carry

            lax.fori_loop(0, PK_CH, pk_r, 0)

        pk_in(0, rows_a, sem_ga).start()
        pk_in(1, rows_b, sem_gb).start()

        def pk_loop(cc, carry):
            for u, (buf, pbuf, sem_g, sem_w) in enumerate(
                    [(rows_a, pout_a, sem_ga, sem_wa),
                     (rows_b, pout_b, sem_gb, sem_wb)]):
                c = 2 * cc + u
                pk_in(c, buf, sem_g).wait()

                @pl.when(cc > 0)
                def _():
                    pk_out(c, pbuf, sem_w).wait()

                pk_pack(buf, pbuf)
                pk_out(c, pbuf, sem_w).start()

                @pl.when(c + 2 < PK_NCH)
                def _():
                    pk_in(c + 2, buf, sem_g).start()

            return carry

        lax.fori_loop(0, (PK_NCH - 1) // 2, pk_loop, 0)
        pk_in(PK_NCH - 1, rows_a, sem_ga).wait()
        pk_out(PK_NCH - 1, pout_a, sem_wa).wait()
        pk_pack(rows_a, pout_a)
        pk_out(PK_NCH - 1, pout_a, sem_wa).start()
        pk_out(0, pout_b, sem_wb).wait()
        pk_out(0, pout_a, sem_wa).wait()

        b0 = wid * G0_PW
        pltpu.async_copy(emb_h.at[idx0_v], rows_a.at[pl.ds(0, G0_PW)], sem_ga).wait()
        pltpu.sync_copy(rows_a.at[pl.ds(0, G0_PW)], g0_h.at[pl.ds(b0, G0_PW)])

        def g1_gather(c, rows, sem):
            return pltpu.make_async_copy(emb_h.at[idx1_v.at[c]], rows, sem)

        def g1_write(c, rows, sem):
            return pltpu.make_async_copy(rows, g1_h.at[pl.ds(b1 + c * CH, CH)], sem)

        g1_gather(0, rows_a, sem_ga).start()
        g1_gather(1, rows_b, sem_gb).start()

        def g1_loop(cc, carry):
            c0 = 2 * cc
            c1 = c0 + 1
            g1_gather(c0, rows_a, sem_ga).wait()
            g1_write(c0, rows_a, sem_wa).start()
            g1_gather(c1, rows_b, sem_gb).wait()
            g1_write(c1, rows_b, sem_wb).start()

            @pl.when(cc < G1_CHUNKS // 2 - 1)
            def _():
                g1_write(c0, rows_a, sem_wa).wait()
                g1_gather(c0 + 2, rows_a, sem_ga).start()
                g1_write(c1, rows_b, sem_wb).wait()
                g1_gather(c1 + 2, rows_b, sem_gb).start()

            return carry

        lax.fori_loop(0, G1_CHUNKS // 2, g1_loop, 0)
        g1_write(0, rows_a, sem_wa).wait()
        g1_write(0, rows_b, sem_wb).wait()

    return k(emb, t0r, t1r)


def _sc_s2(embq, t2r):
    mesh = plsc.VectorSubcoreMesh(core_axis_name="c", subcore_axis_name="s")

    @functools.partial(
        pl.kernel,
        mesh=mesh,
        compiler_params=pltpu.CompilerParams(use_tc_tiling_on_sc=False),
        out_type=jax.ShapeDtypeStruct((M1, D), jnp.float32),
        scratch_types=[
            pltpu.VMEM((S2_CHUNKS, CH), jnp.int32),
            pltpu.VMEM((CH, DW), jnp.int32),
            pltpu.VMEM((CH, DW), jnp.int32),
            pltpu.VMEM((4 * S2_OUT_PC, D), jnp.float32),
            pltpu.VMEM((4 * S2_OUT_PC, D), jnp.float32),
            pltpu.SemaphoreType.DMA,
            pltpu.SemaphoreType.DMA,
            pltpu.SemaphoreType.DMA,
            pltpu.SemaphoreType.DMA,
        ],
    )
    def k(emb_h, t2_h, s2_h,
          idx2_v, rows_a, rows_b, acc_a, acc_b, sem_ga, sem_gb, sem_wa, sem_wb):
        wid = lax.axis_index("s") * NC + lax.axis_index("c")
        b1 = wid * G1_PW

        pltpu.sync_copy(t2_h.at[wid], idx2_v)

        def s2_gather(c, rows, sem):
            return pltpu.make_async_copy(emb_h.at[idx2_v.at[c]], rows, sem)

        def s2_write(c, acc, sem):
            return pltpu.make_async_copy(
                acc, s2_h.at[pl.ds(b1 + c * S2_OUT_PC, 4 * S2_OUT_PC)], sem)

        def _tree(vs):
            while len(vs) > 1:
                vs = [a + b for a, b in zip(vs[::2], vs[1::2])]
            return vs[0]

        def s2_reduce(rows, acc, jbase):
            def red_j(j, carry2):
                base = j * S2
                for lg in range(DW // L):
                    sl = pl.ds(lg * L, L)
                    los, his = [], []
                    for q in range(S2 // 4):
                        lo4, hi4 = [], []
                        for s in range(4 * q, 4 * q + 4):
                            xi = rows[base + s, sl]
                            lo4.append(lax.bitcast_convert_type(xi << 16, jnp.float32))
                            hi4.append(lax.bitcast_convert_type(xi & _HIMASK, jnp.float32))
                        los.append(_tree(lo4))
                        his.append(_tree(hi4))
                    acc[jbase + j, sl] = _tree(los)
                    acc[jbase + j, pl.ds(DW + lg * L, L)] = _tree(his)
                return carry2

            lax.fori_loop(0, S2_OUT_PC, red_j, 0)

        s2_gather(0, rows_a, sem_ga).start()

        NCH8 = S2_CHUNKS // 8

        def s2_loop(cc, carry):
            c0 = 8 * cc
            for u in range(8):
                c = c0 + u
                rows_p, sem_p = (rows_a, sem_ga) if u % 2 == 0 else (rows_b, sem_gb)
                rows_n, sem_n = (rows_b, sem_gb) if u % 2 == 0 else (rows_a, sem_ga)
                acc_p, sem_w = (acc_a, sem_wa) if u < 4 else (acc_b, sem_wb)

                @pl.when(c + 1 < S2_CHUNKS)
                def _():
                    s2_gather(c + 1, rows_n, sem_n).start()

                s2_gather(c, rows_p, sem_p).wait()

                if u % 4 == 0:
                    @pl.when(cc > 0)
                    def _():
                        s2_write(c0, acc_p, sem_w).wait()

                s2_reduce(rows_p, acc_p, (u % 4) * S2_OUT_PC)

                if u % 4 == 3:
                    s2_write(c - 3, acc_p, sem_w).start()

            return carry

        lax.fori_loop(0, NCH8, s2_loop, 0)
        s2_write(0, acc_a, sem_wa).wait()
        s2_write(0, acc_b, sem_wb).wait()

    return k(embq, t2r)


TCB = 2048
NSTEP = M1 // TCB
GPB = TCB // S1


def _tc_body(g0_r, g1_r, s2_r, w1a_r, w1b_r, w1bs_r, w2a_r, w2b_r, wout_r,
             out_r, x1_r, b1_r):
    i = pl.program_id(0)
    g1b = g1_r[...]
    h = jnp.dot(g1b, w1a_r[...], preferred_element_type=jnp.float32)
    h = h + jnp.dot(s2_r[...], w1bs_r[...], preferred_element_type=jnp.float32)
    h = jnp.maximum(h, 0.0)
    b1_r[pl.ds(i * GPB, GPB), :] = h.reshape(GPB, S1, D).sum(axis=1) * (1.0 / S1)
    x1_r[pl.ds(i * GPB, GPB), :] = g1b.reshape(GPB, S1, D).sum(axis=1) * (1.0 / S1)

    @pl.when(i == NSTEP - 1)
    def _():
        a = jnp.dot(g0_r[...], w1a_r[...], preferred_element_type=jnp.float32)
        a = a + jnp.dot(x1_r[...], w1b_r[...], preferred_element_type=jnp.float32)
        a = jnp.maximum(a, 0.0)
        c = jnp.dot(a, w2a_r[...], preferred_element_type=jnp.float32)
        c = c + jnp.dot(b1_r[...], w2b_r[...], preferred_element_type=jnp.float32)
        c = jnp.maximum(c, 0.0)
        out_r[...] = jnp.dot(c, wout_r[...], preferred_element_type=jnp.float32)


def _tc_dense(g0, g1, s2, w1a, w1b, w1bs, w2a, w2b, woutT):
    full = lambda shape: pl.BlockSpec(shape, lambda i: (0, 0))
    return pl.pallas_call(
        _tc_body,
        grid=(NSTEP,),
        in_specs=[
            full((B, D)),
            pl.BlockSpec((TCB, D), lambda i: (i, 0)),
            pl.BlockSpec((TCB, D), lambda i: (i, 0)),
            full((D, D)), full((D, D)), full((D, D)),
            full((D, D)), full((D, D)),
            full((D, NCLS)),
        ],
        out_specs=full((B, NCLS)),
        out_shape=jax.ShapeDtypeStruct((B, NCLS), jnp.float32),
        scratch_shapes=[
            pltpu.VMEM((B, D), jnp.float32),
            pltpu.VMEM((B, D), jnp.float32),
        ],
    )(g0, g1, s2, w1a, w1b, w1bs, w2a, w2b, woutT)


def kernel(T0, T1, T2, emb, W1, W2, Wout):
    t0r = T0.astype(jnp.int32).reshape(B // CH, CH)
    t1r = T1.reshape(-1).astype(jnp.int32).reshape(NW, G1_CHUNKS, CH)
    t2r = T2.reshape(-1).astype(jnp.int32).reshape(NW, S2_CHUNKS, CH)

    w1a = W1[:, :D].T
    w1b = W1[:, D:].T
    w1bs = w1b * (1.0 / S2)
    w2a = W2[:, :D].T
    w2b = W2[:, D:].T
    woutT = Wout.T
    g0, g1, embq = _sc_gather01(emb, t0r, t1r)
    s2 = _sc_s2(embq, t2r)
    return _tc_dense(g0, g1, s2, w1a, w1b, w1bs, w2a, w2b, woutT)
